# Initial kernel scaffold; baseline (speedup 1.0000x reference)
import functools

import jax
import jax.numpy as jnp
from jax import lax
from jax.experimental import pallas as pl
from jax.experimental.pallas import tpu as pltpu
from jax.experimental.pallas import tpu_sc as plsc

N = 16384
D = 256
LAYERS = 3
NC = 2
NS = 16
DC = 4
DCW = D // DC
BLK = 128
SUP = 32
ROWS_PER_TILE = N // NS
SUB = 4
SUB_ROWS = ROWS_PER_TILE // SUB


def _sc_body(nblk_per_tile, cols, rows, vals, xin, fin, xbuf,
             acc, est_c, est_r, est_v, gbuf, zbuf, fbuf, gsem):
    cid = lax.axis_index("c")
    sid = lax.axis_index("s")
    nsup = nblk_per_tile // SUP

    def zero_zbuf():
        zv = jnp.zeros((16,), jnp.float32)

        def zrow(r, _):
            for k in range(DCW // 16):
                zbuf[r, pl.ds(k * 16, 16)] = zv
            return 0
        lax.fori_loop(0, SUB_ROWS, zrow, 0)

    def add_zbuf_into_fbuf():
        def arow(r, _):
            for k in range(DCW // 16):
                sl = pl.ds(k * 16, 16)
                fbuf[r, sl] = fbuf[r, sl] + zbuf[r, sl]
            return 0
        lax.fori_loop(0, SUB_ROWS, arow, 0)

    for cc in range(DC // NC):
        c = cid * (DC // NC) + cc

        for ss in range(SUB):
            r0 = sid * ROWS_PER_TILE + ss * SUB_ROWS
            sl = pl.ds(r0, SUB_ROWS)
            pltpu.sync_copy(xin.at[c].at[sl], fbuf)
            pltpu.sync_copy(fbuf, fin.at[c].at[sl])
            pltpu.sync_copy(fbuf, xbuf.at[c].at[sl])
        plsc.subcore_barrier()

        for _layer in range(LAYERS):
            zero_zbuf()
            for ss in range(SUB):
                r0 = sid * ROWS_PER_TILE + ss * SUB_ROWS
                pltpu.sync_copy(zbuf, acc.at[pl.ds(r0, SUB_ROWS)])
            plsc.subcore_barrier()

            blk_base = sid * nblk_per_tile

            def sup_body(sb, _):
                b0 = blk_base + sb * SUP
                pltpu.sync_copy(cols.at[pl.ds(b0, SUP)], est_c)
                pltpu.sync_copy(rows.at[pl.ds(b0, SUP)], est_r)
                pltpu.sync_copy(vals.at[pl.ds(b0, SUP)], est_v)

                def blk_body(j, _):
                    pltpu.async_copy(xbuf.at[c].at[est_c.at[j]], gbuf,
                                     gsem).wait()

                    def scale(i, _):
                        v = est_v[j, i]
                        for k in range(DCW // 16):
                            sl = pl.ds(k * 16, 16)
                            gbuf[i, sl] = gbuf[i, sl] * v
                        return 0
                    lax.fori_loop(0, BLK, scale, 0)
                    pltpu.sync_copy(gbuf, acc.at[est_r.at[j]], add=True)
                    return 0
                lax.fori_loop(0, SUP, blk_body, 0)
                return 0
            lax.fori_loop(0, nsup, sup_body, 0)
            plsc.subcore_barrier()

            for ss in range(SUB):
                r0 = sid * ROWS_PER_TILE + ss * SUB_ROWS
                sl = pl.ds(r0, SUB_ROWS)
                pltpu.sync_copy(acc.at[pl.ds(r0, SUB_ROWS)], zbuf)
                pltpu.sync_copy(fin.at[c].at[sl], fbuf)
                add_zbuf_into_fbuf()
                pltpu.sync_copy(fbuf, fin.at[c].at[sl])
                pltpu.sync_copy(zbuf, xbuf.at[c].at[sl])
            plsc.subcore_barrier()


def kernel(values, row, col, embedding):
    values = values.astype(jnp.float32)
    row = row.astype(jnp.int32)
    col = col.astype(jnp.int32)
    embedding = embedding.astype(jnp.float32)

    nnz = values.shape[0]
    per_tile_edges = -(-nnz // (NS * SUP * BLK)) * SUP * BLK
    tot = per_tile_edges * NS
    pad = tot - nnz
    values = jnp.pad(values, (0, pad))
    row = jnp.pad(row, (0, pad))
    col = jnp.pad(col, (0, pad))
    cols2d = col.reshape(tot // BLK, BLK)
    rows2d = row.reshape(tot // BLK, BLK)
    vals2d = values.reshape(tot // BLK, BLK)

    xin = jnp.transpose(embedding.reshape(N, DC, DCW), (1, 0, 2))

    nblk_per_tile = per_tile_edges // BLK

    grid_kernel = functools.partial(
        pl.kernel,
        mesh=plsc.VectorSubcoreMesh(core_axis_name="c", subcore_axis_name="s"),
        out_type=(
            jax.ShapeDtypeStruct((DC, N, DCW), jnp.float32),
            jax.ShapeDtypeStruct((DC, N, DCW), jnp.float32),
        ),
        scratch_types=[
            pltpu.VMEM_SHARED((N, DCW), jnp.float32),
            pltpu.VMEM((SUP, BLK), jnp.int32),
            pltpu.VMEM((SUP, BLK), jnp.int32),
            pltpu.VMEM((SUP, BLK), jnp.float32),
            pltpu.VMEM((BLK, DCW), jnp.float32),
            pltpu.VMEM((SUB_ROWS, DCW), jnp.float32),
            pltpu.VMEM((SUB_ROWS, DCW), jnp.float32),
            pltpu.SemaphoreType.DMA,
        ],
    )(functools.partial(_sc_body, nblk_per_tile))

    fin, _ = grid_kernel(cols2d, rows2d, vals2d, xin)
    return jnp.transpose(fin, (1, 0, 2)).reshape(N, D)


# speedup vs baseline: 3.1885x; 3.1885x over previous
import functools

import jax
import jax.numpy as jnp
from jax import lax
from jax.experimental import pallas as pl
from jax.experimental.pallas import tpu as pltpu
from jax.experimental.pallas import tpu_sc as plsc

N = 16384
D = 256
LAYERS = 3
NC = 2
NS = 16
DC = 4
DCW = D // DC
BLK = 128
SUP = 32
ROWS_PER_TILE = N // NS
SUB = 4
SUB_ROWS = ROWS_PER_TILE // SUB


def _sc_body(nblk_per_tile, cols, rows, vals, xin, fin, xbuf,
             acc, est_c, est_r, est_v, gbuf, zbuf, fbuf, gsem):
    cid = lax.axis_index("c")
    sid = lax.axis_index("s")
    nsup = nblk_per_tile // SUP

    def zero_zbuf():
        zv = jnp.zeros((16,), jnp.float32)

        def zrow(r, _):
            for k in range(DCW // 16):
                zbuf[r, pl.ds(k * 16, 16)] = zv
            return 0
        lax.fori_loop(0, SUB_ROWS, zrow, 0)

    def add_zbuf_into_fbuf():
        def arow(r, _):
            for k in range(DCW // 16):
                sl = pl.ds(k * 16, 16)
                fbuf[r, sl] = fbuf[r, sl] + zbuf[r, sl]
            return 0
        lax.fori_loop(0, SUB_ROWS, arow, 0)

    for cc in range(DC // NC):
        c = cid * (DC // NC) + cc

        for ss in range(SUB):
            r0 = sid * ROWS_PER_TILE + ss * SUB_ROWS
            sl = pl.ds(r0, SUB_ROWS)
            pltpu.sync_copy(xin.at[c].at[sl], fbuf)
            pltpu.sync_copy(fbuf, fin.at[c].at[sl])
            pltpu.sync_copy(fbuf, xbuf.at[c].at[sl])
        plsc.subcore_barrier()

        for _layer in range(LAYERS):
            zero_zbuf()
            for ss in range(SUB):
                r0 = sid * ROWS_PER_TILE + ss * SUB_ROWS
                pltpu.sync_copy(zbuf, acc.at[pl.ds(r0, SUB_ROWS)])
            plsc.subcore_barrier()

            blk_base = sid * nblk_per_tile

            def sup_body(sb, _):
                b0 = blk_base + sb * SUP
                pltpu.sync_copy(cols.at[pl.ds(b0, SUP)], est_c)
                pltpu.sync_copy(rows.at[pl.ds(b0, SUP)], est_r)
                pltpu.sync_copy(vals.at[pl.ds(b0, SUP)], est_v)

                def blk_body(j, _):
                    pltpu.async_copy(xbuf.at[c].at[est_c.at[j]], gbuf,
                                     gsem).wait()

                    def scale(m, _):
                        vv = est_v[j, pl.ds(m * 16, 16)]
                        base = m * 16
                        for t in range(16):
                            v = vv[t]
                            i = base + t
                            for k in range(DCW // 16):
                                sl = pl.ds(k * 16, 16)
                                gbuf[i, sl] = gbuf[i, sl] * v
                        return 0
                    lax.fori_loop(0, BLK // 16, scale, 0)
                    pltpu.sync_copy(gbuf, acc.at[est_r.at[j]], add=True)
                    return 0
                lax.fori_loop(0, SUP, blk_body, 0)
                return 0
            lax.fori_loop(0, nsup, sup_body, 0)
            plsc.subcore_barrier()

            for ss in range(SUB):
                r0 = sid * ROWS_PER_TILE + ss * SUB_ROWS
                sl = pl.ds(r0, SUB_ROWS)
                pltpu.sync_copy(acc.at[pl.ds(r0, SUB_ROWS)], zbuf)
                pltpu.sync_copy(fin.at[c].at[sl], fbuf)
                add_zbuf_into_fbuf()
                pltpu.sync_copy(fbuf, fin.at[c].at[sl])
                pltpu.sync_copy(zbuf, xbuf.at[c].at[sl])
            plsc.subcore_barrier()


def kernel(values, row, col, embedding):
    values = values.astype(jnp.float32)
    row = row.astype(jnp.int32)
    col = col.astype(jnp.int32)
    embedding = embedding.astype(jnp.float32)

    nnz = values.shape[0]
    per_tile_edges = -(-nnz // (NS * SUP * BLK)) * SUP * BLK
    tot = per_tile_edges * NS
    pad = tot - nnz
    values = jnp.pad(values, (0, pad))
    row = jnp.pad(row, (0, pad))
    col = jnp.pad(col, (0, pad))
    cols2d = col.reshape(tot // BLK, BLK)
    rows2d = row.reshape(tot // BLK, BLK)
    vals2d = values.reshape(tot // BLK, BLK)

    xin = jnp.transpose(embedding.reshape(N, DC, DCW), (1, 0, 2))

    nblk_per_tile = per_tile_edges // BLK

    grid_kernel = functools.partial(
        pl.kernel,
        mesh=plsc.VectorSubcoreMesh(core_axis_name="c", subcore_axis_name="s"),
        compiler_params=pltpu.CompilerParams(use_tc_tiling_on_sc=False),
        out_type=(
            jax.ShapeDtypeStruct((DC, N, DCW), jnp.float32),
            jax.ShapeDtypeStruct((DC, N, DCW), jnp.float32),
        ),
        scratch_types=[
            pltpu.VMEM_SHARED((N, DCW), jnp.float32),
            pltpu.VMEM((SUP, BLK), jnp.int32),
            pltpu.VMEM((SUP, BLK), jnp.int32),
            pltpu.VMEM((SUP, BLK), jnp.float32),
            pltpu.VMEM((BLK, DCW), jnp.float32),
            pltpu.VMEM((SUB_ROWS, DCW), jnp.float32),
            pltpu.VMEM((SUB_ROWS, DCW), jnp.float32),
            pltpu.SemaphoreType.DMA,
        ],
    )(functools.partial(_sc_body, nblk_per_tile))

    fin, _ = grid_kernel(cols2d, rows2d, vals2d, xin)
    return jnp.transpose(fin, (1, 0, 2)).reshape(N, D)


# speedup vs baseline: 5.0351x; 1.5792x over previous
import functools

import jax
import jax.numpy as jnp
from jax import lax
from jax.experimental import pallas as pl
from jax.experimental.pallas import tpu as pltpu
from jax.experimental.pallas import tpu_sc as plsc

N = 16384
D = 256
LAYERS = 3
NC = 2
NS = 16
DC = 4
DCW = D // DC
BLK = 128
SUP = 32
NBUF = 4
ROWS_PER_TILE = N // NS
SUB = 16
SUB_ROWS = ROWS_PER_TILE // SUB


def _sc_body(nblk_per_tile, edges, vals, xin, fin, xbuf,
             acc, est, est_v, g0, g1, g2, g3, zbuf, fbuf,
             gs0, gs1, gs2, gs3, ss0, ss1, ss2, ss3):
    cid = lax.axis_index("c")
    sid = lax.axis_index("s")
    gbufs = (g0, g1, g2, g3)
    gsems = (gs0, gs1, gs2, gs3)
    ssems = (ss0, ss1, ss2, ss3)
    nquad = nblk_per_tile // NBUF
    nsup = nblk_per_tile // SUP
    quads_per_sup = SUP // NBUF

    def zero_zbuf():
        zv = jnp.zeros((16,), jnp.float32)

        def zrow(r, _):
            for k in range(DCW // 16):
                zbuf[r, pl.ds(k * 16, 16)] = zv
            return 0
        lax.fori_loop(0, SUB_ROWS, zrow, 0)

    def add_zbuf_into_fbuf():
        def arow(r, _):
            for k in range(DCW // 16):
                sl = pl.ds(k * 16, 16)
                fbuf[r, sl] = fbuf[r, sl] + zbuf[r, sl]
            return 0
        lax.fori_loop(0, SUB_ROWS, arow, 0)

    def chunk_body(cc, _):
        c = cid * (DC // NC) + cc

        def init_ss(ss, _):
            r0 = sid * ROWS_PER_TILE + ss * SUB_ROWS
            sl = pl.ds(r0, SUB_ROWS)
            pltpu.sync_copy(xin.at[c].at[sl], fbuf)
            pltpu.sync_copy(fbuf, fin.at[c].at[sl])
            pltpu.sync_copy(fbuf, xbuf.at[c].at[sl])
            return 0
        lax.fori_loop(0, SUB, init_ss, 0)
        plsc.subcore_barrier()

        def layer_body(_layer, __):
            zero_zbuf()

            def zero_ss(ss, _):
                r0 = sid * ROWS_PER_TILE + ss * SUB_ROWS
                pltpu.sync_copy(zbuf, acc.at[pl.ds(r0, SUB_ROWS)])
                return 0
            lax.fori_loop(0, SUB, zero_ss, 0)
            plsc.subcore_barrier()

            blk_base = sid * nblk_per_tile

            def stage(sup_idx, buf_idx):
                pltpu.sync_copy(edges.at[pl.ds(blk_base + sup_idx * SUP, SUP)],
                                est.at[buf_idx])
                pltpu.sync_copy(vals.at[pl.ds(blk_base + sup_idx * SUP, SUP)],
                                est_v.at[buf_idx])

            def start_gather(par, j, t):
                pltpu.async_copy(xbuf.at[c].at[est.at[par, j, 0]],
                                 gbufs[t], gsems[t])

            def start_scatter(par, j, t):
                pltpu.async_copy(gbufs[t], acc.at[est.at[par, j, 1]],
                                 ssems[t], add=True)

            def scale(par, j, t):
                g = gbufs[t]

                def m_body(m, _):
                    vv = est_v[par, j, pl.ds(m * 16, 16)]
                    base = m * 16
                    for tt in range(16):
                        v = vv[tt]
                        i = base + tt
                        for k in range(DCW // 16):
                            sl = pl.ds(k * 16, 16)
                            g[i, sl] = g[i, sl] * v
                    return 0
                lax.fori_loop(0, BLK // 16, m_body, 0)

            stage(0, 0)
            start_gather(0, 0, 0)
            start_gather(0, 1, 1)

            def quad(p, _):
                for t in range(NBUF):
                    k = p * NBUF + t
                    j = jnp.bitwise_and(k, SUP - 1)
                    par = jnp.bitwise_and(jnp.right_shift(k, 5), 1)
                    t2 = (t + 2) % NBUF
                    pltpu.make_async_copy(
                        xbuf.at[c].at[est.at[par, j, 0]], gbufs[t],
                        gsems[t]).wait()
                    scale(par, j, t)
                    start_scatter(par, j, t)
                    if t < 2:
                        @pl.when(p > 0)
                        def _():
                            pltpu.make_async_copy(
                                gbufs[t2], acc.at[est.at[par, j, 1]],
                                ssems[t2]).wait()
                    else:
                        pltpu.make_async_copy(
                            gbufs[t2], acc.at[est.at[par, j, 1]],
                            ssems[t2]).wait()
                    if t == 2:
                        @pl.when(jnp.logical_and(
                            jnp.bitwise_and(p, quads_per_sup - 1)
                            == quads_per_sup - 1,
                            p < nquad - 1))
                        def _():
                            stage(jnp.right_shift(p, 3) + 1,
                                  jnp.bitwise_and(jnp.right_shift(p, 3) + 1,
                                                  1))
                    kn = k + 2
                    jn = jnp.bitwise_and(kn, SUP - 1)
                    parn = jnp.bitwise_and(jnp.right_shift(kn, 5), 1)
                    if t < 2:
                        start_gather(parn, jn, t2)
                    else:
                        @pl.when(p < nquad - 1)
                        def _():
                            start_gather(parn, jn, t2)
                return 0
            lax.fori_loop(0, nquad, quad, 0)
            pltpu.make_async_copy(gbufs[2], acc.at[est.at[0, 0, 1]],
                                  ssems[2]).wait()
            pltpu.make_async_copy(gbufs[3], acc.at[est.at[0, 0, 1]],
                                  ssems[3]).wait()
            plsc.subcore_barrier()

            def end_ss(ss, _):
                r0 = sid * ROWS_PER_TILE + ss * SUB_ROWS
                sl = pl.ds(r0, SUB_ROWS)
                pltpu.sync_copy(acc.at[pl.ds(r0, SUB_ROWS)], zbuf)
                pltpu.sync_copy(fin.at[c].at[sl], fbuf)
                add_zbuf_into_fbuf()
                pltpu.sync_copy(fbuf, fin.at[c].at[sl])
                pltpu.sync_copy(zbuf, xbuf.at[c].at[sl])
                return 0
            lax.fori_loop(0, SUB, end_ss, 0)
            plsc.subcore_barrier()
            return 0
        lax.fori_loop(0, LAYERS, layer_body, 0)
        return 0
    lax.fori_loop(0, DC // NC, chunk_body, 0)


def kernel(values, row, col, embedding):
    values = values.astype(jnp.float32)
    row = row.astype(jnp.int32)
    col = col.astype(jnp.int32)
    embedding = embedding.astype(jnp.float32)

    nnz = values.shape[0]
    per_tile_edges = -(-nnz // (NS * SUP * BLK)) * SUP * BLK
    tot = per_tile_edges * NS
    pad = tot - nnz
    values = jnp.pad(values, (0, pad))
    row = jnp.pad(row, (0, pad))
    col = jnp.pad(col, (0, pad))
    edges = jnp.stack(
        [col.reshape(tot // BLK, BLK),
         row.reshape(tot // BLK, BLK)],
        axis=1)
    vals2d = values.reshape(tot // BLK, BLK)

    xin = jnp.transpose(embedding.reshape(N, DC, DCW), (1, 0, 2))

    nblk_per_tile = per_tile_edges // BLK

    grid_kernel = functools.partial(
        pl.kernel,
        mesh=plsc.VectorSubcoreMesh(core_axis_name="c", subcore_axis_name="s"),
        compiler_params=pltpu.CompilerParams(use_tc_tiling_on_sc=False),
        out_type=(
            jax.ShapeDtypeStruct((DC, N, DCW), jnp.float32),
            jax.ShapeDtypeStruct((DC, N, DCW), jnp.float32),
        ),
        scratch_types=[
            pltpu.VMEM_SHARED((N, DCW), jnp.float32),
            pltpu.VMEM((2, SUP, 2, BLK), jnp.int32),
            pltpu.VMEM((2, SUP, BLK), jnp.float32),
            pltpu.VMEM((BLK, DCW), jnp.float32),
            pltpu.VMEM((BLK, DCW), jnp.float32),
            pltpu.VMEM((BLK, DCW), jnp.float32),
            pltpu.VMEM((BLK, DCW), jnp.float32),
            pltpu.VMEM((SUB_ROWS, DCW), jnp.float32),
            pltpu.VMEM((SUB_ROWS, DCW), jnp.float32),
            pltpu.SemaphoreType.DMA,
            pltpu.SemaphoreType.DMA,
            pltpu.SemaphoreType.DMA,
            pltpu.SemaphoreType.DMA,
            pltpu.SemaphoreType.DMA,
            pltpu.SemaphoreType.DMA,
            pltpu.SemaphoreType.DMA,
            pltpu.SemaphoreType.DMA,
        ],
    )(functools.partial(_sc_body, nblk_per_tile))

    fin, _ = grid_kernel(edges, vals2d, xin)
    return jnp.transpose(fin, (1, 0, 2)).reshape(N, D)


# speedup vs baseline: 13.4284x; 2.6670x over previous
import functools

import jax
import jax.numpy as jnp
from jax import lax
from jax.experimental import pallas as pl
from jax.experimental.pallas import tpu as pltpu
from jax.experimental.pallas import tpu_sc as plsc

N = 16384
D = 256
LAYERS = 3
NC = 2
NS = 16
DC = 4
DCW = D // DC
BLK = 128
SUP = 32
NBUF = 4
ROWS_PER_TILE = N // NS
SUB = 16
SUB_ROWS = ROWS_PER_TILE // SUB


def _sc_body(nblk_per_tile, edges, vals, xin, fin, xbuf,
             acc, est, est_v, g0, g1, g2, g3, zbuf, fbuf,
             gs0, gs1, gs2, gs3, ss0, ss1, ss2, ss3):
    cid = lax.axis_index("c")
    sid = lax.axis_index("s")
    gbufs = (g0, g1, g2, g3)
    gsems = (gs0, gs1, gs2, gs3)
    ssems = (ss0, ss1, ss2, ss3)
    nquad = nblk_per_tile // NBUF
    nsup = nblk_per_tile // SUP
    quads_per_sup = SUP // NBUF

    def zero_zbuf():
        zv = jnp.zeros((16,), jnp.float32)

        def zrow(r, _):
            for k in range(DCW // 16):
                zbuf[r, pl.ds(k * 16, 16)] = zv
            return 0
        lax.fori_loop(0, SUB_ROWS, zrow, 0)

    def add_zbuf_into_fbuf():
        def arow(r, _):
            for k in range(DCW // 16):
                sl = pl.ds(k * 16, 16)
                fbuf[r, sl] = fbuf[r, sl] + zbuf[r, sl]
            return 0
        lax.fori_loop(0, SUB_ROWS, arow, 0)

    def chunk_body(cc, _):
        c = cid * (DC // NC) + cc

        def init_ss(ss, _):
            r0 = sid * ROWS_PER_TILE + ss * SUB_ROWS
            sl = pl.ds(r0, SUB_ROWS)
            pltpu.sync_copy(xin.at[c].at[sl], fbuf)
            pltpu.sync_copy(fbuf, fin.at[c].at[sl])
            pltpu.sync_copy(fbuf, xbuf.at[c].at[sl])
            return 0
        lax.fori_loop(0, SUB, init_ss, 0)
        plsc.subcore_barrier()

        def layer_body(_layer, __):
            zero_zbuf()

            def zero_ss(ss, _):
                r0 = sid * ROWS_PER_TILE + ss * SUB_ROWS
                pltpu.sync_copy(zbuf, acc.at[pl.ds(r0, SUB_ROWS)])
                return 0
            lax.fori_loop(0, SUB, zero_ss, 0)
            plsc.subcore_barrier()

            blk_base = sid * nblk_per_tile

            def stage(sup_idx, buf_idx):
                pltpu.sync_copy(edges.at[pl.ds(blk_base + sup_idx * SUP, SUP)],
                                est.at[buf_idx])
                pltpu.sync_copy(vals.at[pl.ds(blk_base + sup_idx * SUP, SUP)],
                                est_v.at[buf_idx])

            def start_gather(par, j, t):
                pltpu.async_copy(xbuf.at[c].at[est.at[par, j, 0]],
                                 gbufs[t], gsems[t])

            def start_scatter(par, j, t):
                pltpu.async_copy(gbufs[t], acc.at[est.at[par, j, 1]],
                                 ssems[t], add=True)

            def scale(par, j, t):
                g = gbufs[t]

                @plsc.parallel_loop(0, BLK // 16, 1, unroll=2)
                def m_body(m):
                    vv = est_v[par, j, pl.ds(m * 16, 16)]
                    base = m * 16
                    for tt in range(16):
                        v = vv[tt]
                        i = base + tt
                        for k in range(DCW // 16):
                            sl = pl.ds(k * 16, 16)
                            g[i, sl] = g[i, sl] * v

            stage(0, 0)
            start_gather(0, 0, 0)
            start_gather(0, 1, 1)

            def quad(p, _):
                for t in range(NBUF):
                    k = p * NBUF + t
                    j = jnp.bitwise_and(k, SUP - 1)
                    par = jnp.bitwise_and(jnp.right_shift(k, 5), 1)
                    t2 = (t + 2) % NBUF
                    pltpu.make_async_copy(
                        xbuf.at[c].at[est.at[par, j, 0]], gbufs[t],
                        gsems[t]).wait()
                    scale(par, j, t)
                    start_scatter(par, j, t)
                    if t < 2:
                        @pl.when(p > 0)
                        def _():
                            pltpu.make_async_copy(
                                gbufs[t2], acc.at[est.at[par, j, 1]],
                                ssems[t2]).wait()
                    else:
                        pltpu.make_async_copy(
                            gbufs[t2], acc.at[est.at[par, j, 1]],
                            ssems[t2]).wait()
                    if t == 2:
                        @pl.when(jnp.logical_and(
                            jnp.bitwise_and(p, quads_per_sup - 1)
                            == quads_per_sup - 1,
                            p < nquad - 1))
                        def _():
                            stage(jnp.right_shift(p, 3) + 1,
                                  jnp.bitwise_and(jnp.right_shift(p, 3) + 1,
                                                  1))
                    kn = k + 2
                    jn = jnp.bitwise_and(kn, SUP - 1)
                    parn = jnp.bitwise_and(jnp.right_shift(kn, 5), 1)
                    if t < 2:
                        start_gather(parn, jn, t2)
                    else:
                        @pl.when(p < nquad - 1)
                        def _():
                            start_gather(parn, jn, t2)
                return 0
            lax.fori_loop(0, nquad, quad, 0)
            pltpu.make_async_copy(gbufs[2], acc.at[est.at[0, 0, 1]],
                                  ssems[2]).wait()
            pltpu.make_async_copy(gbufs[3], acc.at[est.at[0, 0, 1]],
                                  ssems[3]).wait()
            plsc.subcore_barrier()

            def end_ss(ss, _):
                r0 = sid * ROWS_PER_TILE + ss * SUB_ROWS
                sl = pl.ds(r0, SUB_ROWS)
                pltpu.sync_copy(acc.at[pl.ds(r0, SUB_ROWS)], zbuf)
                pltpu.sync_copy(fin.at[c].at[sl], fbuf)
                add_zbuf_into_fbuf()
                pltpu.sync_copy(fbuf, fin.at[c].at[sl])
                pltpu.sync_copy(zbuf, xbuf.at[c].at[sl])
                return 0
            lax.fori_loop(0, SUB, end_ss, 0)
            plsc.subcore_barrier()
            return 0
        lax.fori_loop(0, LAYERS, layer_body, 0)
        return 0
    lax.fori_loop(0, DC // NC, chunk_body, 0)


def kernel(values, row, col, embedding):
    values = values.astype(jnp.float32)
    row = row.astype(jnp.int32)
    col = col.astype(jnp.int32)
    embedding = embedding.astype(jnp.float32)

    nnz = values.shape[0]
    per_tile_edges = -(-nnz // (NS * SUP * BLK)) * SUP * BLK
    tot = per_tile_edges * NS
    pad = tot - nnz
    values = jnp.pad(values, (0, pad))
    row = jnp.pad(row, (0, pad))
    col = jnp.pad(col, (0, pad))
    edges = jnp.stack(
        [col.reshape(tot // BLK, BLK),
         row.reshape(tot // BLK, BLK)],
        axis=1)
    vals2d = values.reshape(tot // BLK, BLK)

    xin = jnp.transpose(embedding.reshape(N, DC, DCW), (1, 0, 2))

    nblk_per_tile = per_tile_edges // BLK

    grid_kernel = functools.partial(
        pl.kernel,
        mesh=plsc.VectorSubcoreMesh(core_axis_name="c", subcore_axis_name="s"),
        compiler_params=pltpu.CompilerParams(use_tc_tiling_on_sc=False),
        out_type=(
            jax.ShapeDtypeStruct((DC, N, DCW), jnp.float32),
            jax.ShapeDtypeStruct((DC, N, DCW), jnp.float32),
        ),
        scratch_types=[
            pltpu.VMEM_SHARED((N, DCW), jnp.float32),
            pltpu.VMEM((2, SUP, 2, BLK), jnp.int32),
            pltpu.VMEM((2, SUP, BLK), jnp.float32),
            pltpu.VMEM((BLK, DCW), jnp.float32),
            pltpu.VMEM((BLK, DCW), jnp.float32),
            pltpu.VMEM((BLK, DCW), jnp.float32),
            pltpu.VMEM((BLK, DCW), jnp.float32),
            pltpu.VMEM((SUB_ROWS, DCW), jnp.float32),
            pltpu.VMEM((SUB_ROWS, DCW), jnp.float32),
            pltpu.SemaphoreType.DMA,
            pltpu.SemaphoreType.DMA,
            pltpu.SemaphoreType.DMA,
            pltpu.SemaphoreType.DMA,
            pltpu.SemaphoreType.DMA,
            pltpu.SemaphoreType.DMA,
            pltpu.SemaphoreType.DMA,
            pltpu.SemaphoreType.DMA,
        ],
    )(functools.partial(_sc_body, nblk_per_tile))

    fin, _ = grid_kernel(edges, vals2d, xin)
    return jnp.transpose(fin, (1, 0, 2)).reshape(N, D)
